# baseline (speedup 1.0000x reference)
import functools

import jax
import jax.numpy as jnp
from jax import lax
from jax.experimental import pallas as pl
from jax.experimental.pallas import tpu as pltpu
from jax.experimental.pallas import tpu_sc as plsc

N = 10000
H = 128
E = 320000

NUM_SC = 2
NUM_TILES = 16
CHUNK = 256
CHUNKS = 80
E_PAD = NUM_TILES * CHUNKS * CHUNK
N_PAD = 10112
ROWS_PER_TILE = N_PAD // NUM_TILES
N_ACC = N_PAD

ROW_BLK = 400
MM_BLK = 1000

_MESH = plsc.VectorSubcoreMesh(core_axis_name="c", subcore_axis_name="s")


def _sc_deg_body(sidx_hbm, ones_hbm, zeros_hbm, out_hbm, sidx_v, ones_v, acc):
    c = lax.axis_index("c")
    s = lax.axis_index("s")
    w = c * NUM_TILES + s
    pltpu.sync_copy(sidx_hbm.at[w], sidx_v)
    pltpu.sync_copy(ones_hbm, ones_v)
    base = s * ROWS_PER_TILE
    pltpu.sync_copy(zeros_hbm.at[pl.ds(base, ROWS_PER_TILE)],
                    acc.at[pl.ds(base, ROWS_PER_TILE)])
    plsc.subcore_barrier()

    @pl.loop(0, CHUNKS)
    def _(j):
        pltpu.sync_copy(ones_v, acc.at[sidx_v.at[j]], add=True)

    plsc.subcore_barrier()
    pltpu.sync_copy(acc.at[pl.ds(base, ROWS_PER_TILE)],
                    out_hbm.at[c].at[pl.ds(base, ROWS_PER_TILE)])


def _sc_deg(sidx, ones, zeros):
    k = pl.kernel(
        _sc_deg_body,
        out_type=jax.ShapeDtypeStruct((NUM_SC, N_PAD, 16), jnp.float32),
        mesh=_MESH,
        scratch_types=[
            pltpu.VMEM((CHUNKS, CHUNK), jnp.int32),
            pltpu.VMEM((CHUNK, 16), jnp.float32),
            pltpu.VMEM_SHARED((N_ACC, 16), jnp.float32),
        ],
        compiler_params=pltpu.CompilerParams(use_tc_tiling_on_sc=False),
    )
    return k(sidx, ones, zeros)


HH = H // 2


NBUF = 2


def _sc_agg_body(g_hbm, gidx_hbm, sidx_hbm, out_hbm, gidx_v, sidx_v,
                 b0, b1, acc,
                 gs0, gs1, ss0, ss1):
    bufs = (b0, b1)
    gsems = (gs0, gs1)
    ssems = (ss0, ss1)
    c = lax.axis_index("c")
    s = lax.axis_index("s")
    w = c * NUM_TILES + s
    pltpu.sync_copy(gidx_hbm.at[w], gidx_v)
    pltpu.sync_copy(sidx_hbm.at[w], sidx_v)
    base = s * ROWS_PER_TILE

    for half in range(2):
        pltpu.sync_copy(g_hbm.at[half].at[pl.ds(c * N_PAD + base,
                                                ROWS_PER_TILE)],
                        acc.at[pl.ds(base, ROWS_PER_TILE)])
        plsc.subcore_barrier()

        def g_src(j):
            return g_hbm.at[half].at[gidx_v.at[j]]

        def s_dst(j):
            return acc.at[sidx_v.at[j]]

        for k in range(NBUF):
            pltpu.async_copy(g_src(k), bufs[k], gsems[k])

        @pl.loop(0, CHUNKS // NBUF - 1)
        def _(t):
            j0 = t * NBUF
            for k in range(NBUF):
                pltpu.make_async_copy(g_src(j0 + k), bufs[k], gsems[k]).wait()
                pltpu.async_copy(bufs[k], s_dst(j0 + k), ssems[k], add=True)
            for k in range(NBUF):
                pltpu.make_async_copy(bufs[k], s_dst(j0 + k), ssems[k]).wait()
                pltpu.async_copy(g_src(j0 + NBUF + k), bufs[k], gsems[k])

        j0 = CHUNKS - NBUF
        for k in range(NBUF):
            pltpu.make_async_copy(g_src(j0 + k), bufs[k], gsems[k]).wait()
            pltpu.async_copy(bufs[k], s_dst(j0 + k), ssems[k], add=True)
        for k in range(NBUF):
            pltpu.make_async_copy(bufs[k], s_dst(j0 + k), ssems[k]).wait()

        plsc.subcore_barrier()
        pltpu.sync_copy(acc.at[pl.ds(base, ROWS_PER_TILE)],
                        out_hbm.at[half].at[pl.ds(c * N_PAD + base,
                                                  ROWS_PER_TILE)])


def _sc_agg(g, gidx, sidx):
    k = pl.kernel(
        _sc_agg_body,
        out_type=jax.ShapeDtypeStruct((2, NUM_SC * N_PAD, HH), jnp.float32),
        mesh=_MESH,
        scratch_types=[
            pltpu.VMEM((CHUNKS, CHUNK), jnp.int32),
            pltpu.VMEM((CHUNKS, CHUNK), jnp.int32),
        ] + [pltpu.VMEM((CHUNK, HH), jnp.float32)] * NBUF + [
            pltpu.VMEM_SHARED((N_ACC, HH), jnp.float32),
        ] + [pltpu.SemaphoreType.DMA] * (2 * NBUF),
        compiler_params=pltpu.CompilerParams(use_tc_tiling_on_sc=False),
    )
    return k(g, gidx, sidx)


def _to_halves(gp):
    return gp.reshape(NUM_SC, N_PAD, 2, HH).transpose(2, 0, 1, 3).reshape(
        2, NUM_SC * N_PAD, HH)


def _from_halves(o):
    return o.reshape(2, NUM_SC, N_PAD, HH).transpose(1, 2, 0, 3).reshape(
        NUM_SC, N_PAD, H)[:, :N]


def _mm_shared_body(a_ref, w_ref, o_ref):
    o_ref[...] = jnp.dot(a_ref[...], w_ref[0],
                         preferred_element_type=jnp.float32)[None]


def _mm_shared(x, wstack):
    return pl.pallas_call(
        _mm_shared_body,
        grid=(NUM_SC, N // MM_BLK),
        in_specs=[
            pl.BlockSpec((MM_BLK, H), lambda d, i: (i, 0)),
            pl.BlockSpec((1, H, H), lambda d, i: (d, 0, 0)),
        ],
        out_specs=pl.BlockSpec((1, MM_BLK, H), lambda d, i: (d, i, 0)),
        out_shape=jax.ShapeDtypeStruct((NUM_SC, N, H), jnp.float32),
    )(x, wstack)


def _mm_stacked_body(a_ref, w_ref, o_ref):
    o_ref[...] = jnp.dot(a_ref[0], w_ref[0],
                         preferred_element_type=jnp.float32)[None]


def _mm_stacked(a, wstack):
    return pl.pallas_call(
        _mm_stacked_body,
        grid=(NUM_SC, N // MM_BLK),
        in_specs=[
            pl.BlockSpec((1, MM_BLK, H), lambda d, i: (d, i, 0)),
            pl.BlockSpec((1, H, H), lambda d, i: (d, 0, 0)),
        ],
        out_specs=pl.BlockSpec((1, MM_BLK, H), lambda d, i: (d, i, 0)),
        out_shape=jax.ShapeDtypeStruct((NUM_SC, N, H), jnp.float32),
    )(a, wstack)


def _fc_logsoftmax_body(xf_ref, xb_ref, wf_ref, wb_ref, bfc_ref, out_ref):
    l = jnp.dot(xf_ref[...], wf_ref[...], preferred_element_type=jnp.float32)
    l = l + jnp.dot(xb_ref[...], wb_ref[...], preferred_element_type=jnp.float32)
    l = l + bfc_ref[...]
    m = jnp.max(l, axis=1, keepdims=True)
    lse = m + jnp.log(jnp.sum(jnp.exp(l - m), axis=1, keepdims=True))
    out_ref[...] = l - lse


def _fc_logsoftmax(xf, xb, Wfc, bfc):
    wf = Wfc[:, :H].T
    wb = Wfc[:, H:].T
    b2 = bfc.reshape(1, N)
    return pl.pallas_call(
        _fc_logsoftmax_body,
        grid=(N // ROW_BLK,),
        in_specs=[
            pl.BlockSpec((ROW_BLK, H), lambda i: (i, 0)),
            pl.BlockSpec((ROW_BLK, H), lambda i: (i, 0)),
            pl.BlockSpec((H, N), lambda i: (0, 0)),
            pl.BlockSpec((H, N), lambda i: (0, 0)),
            pl.BlockSpec((1, N), lambda i: (0, 0)),
        ],
        out_specs=pl.BlockSpec((ROW_BLK, N), lambda i: (i, 0)),
        out_shape=jax.ShapeDtypeStruct((N, N), jnp.float32),
    )(xf, xb, wf, wb, b2)


def kernel(x, edge_index, W1f, b1f, W2f, b2f, W1b, b1b, W2b, b2b, Wfc, bfc):
    src = edge_index[0]
    dst = edge_index[1]

    pad = E_PAD - E
    padz = jnp.zeros((pad,), jnp.int32)
    padn = jnp.full((pad,), N, jnp.int32)
    gf = jnp.concatenate([src, padz])
    sf = jnp.concatenate([dst, padn])
    gb = jnp.concatenate([dst + N_PAD, padz])
    sb = jnp.concatenate([src, padn])
    gidx = jnp.stack([gf, gb]).reshape(NUM_SC * NUM_TILES, CHUNKS, CHUNK)
    sidx = jnp.stack([sf, sb]).reshape(NUM_SC * NUM_TILES, CHUNKS, CHUNK)

    ones = jnp.ones((CHUNK, 16), jnp.float32)
    zeros = jnp.zeros((N_PAD, 16), jnp.float32)
    degp = _sc_deg(sidx, ones, zeros)
    dinv = lax.rsqrt(degp[:, :N, 0] + 1.0)
    dinv3 = dinv[:, :, None]

    wstack1 = jnp.stack([W1f.T, W1b.T])
    wstack2 = jnp.stack([W2f.T, W2b.T])
    bstack1 = jnp.stack([b1f, b1b])[:, None, :]
    bstack2 = jnp.stack([b2f, b2b])[:, None, :]

    rowpad = ((0, 0), (0, N_PAD - N), (0, 0))
    h1 = _mm_shared(x, wstack1)
    g1 = _to_halves(jnp.pad(h1 * dinv3, rowpad))
    agg1 = _from_halves(_sc_agg(g1, gidx, sidx))
    x1 = jax.nn.relu(agg1 * dinv3 + bstack1)

    h2 = _mm_stacked(x1, wstack2)
    g2 = _to_halves(jnp.pad(h2 * dinv3, rowpad))
    agg2 = _from_halves(_sc_agg(g2, gidx, sidx))
    x2 = jax.nn.relu(agg2 * dinv3 + bstack2)

    return _fc_logsoftmax(x2[0], x2[1], Wfc, bfc)


# speedup vs baseline: 1.7641x; 1.7641x over previous
import functools

import jax
import jax.numpy as jnp
from jax import lax
from jax.experimental import pallas as pl
from jax.experimental.pallas import tpu as pltpu
from jax.experimental.pallas import tpu_sc as plsc

N = 10000
H = 128
E = 320000

NUM_SC = 2
NUM_TILES = 16
CHUNK = 128
CHUNKS = 160
E_PAD = NUM_TILES * CHUNKS * CHUNK
N_PAD = 10112
ROWS_PER_TILE = N_PAD // NUM_TILES
N_ACC = N_PAD

ROW_BLK = 400
MM_BLK = 1000

_MESH = plsc.VectorSubcoreMesh(core_axis_name="c", subcore_axis_name="s")


def _sc_deg_body(sidx_hbm, ones_hbm, zeros_hbm, out_hbm, sidx_v, ones_v, acc):
    c = lax.axis_index("c")
    s = lax.axis_index("s")
    w = c * NUM_TILES + s
    pltpu.sync_copy(sidx_hbm.at[w], sidx_v)
    pltpu.sync_copy(ones_hbm, ones_v)
    base = s * ROWS_PER_TILE
    pltpu.sync_copy(zeros_hbm.at[pl.ds(base, ROWS_PER_TILE)],
                    acc.at[pl.ds(base, ROWS_PER_TILE)])
    plsc.subcore_barrier()

    @pl.loop(0, CHUNKS)
    def _(j):
        pltpu.sync_copy(ones_v, acc.at[sidx_v.at[j]], add=True)

    plsc.subcore_barrier()
    pltpu.sync_copy(acc.at[pl.ds(base, ROWS_PER_TILE)],
                    out_hbm.at[c].at[pl.ds(base, ROWS_PER_TILE)])


def _sc_deg(sidx, ones, zeros):
    k = pl.kernel(
        _sc_deg_body,
        out_type=jax.ShapeDtypeStruct((NUM_SC, N_PAD, 16), jnp.float32),
        mesh=_MESH,
        scratch_types=[
            pltpu.VMEM((CHUNKS, CHUNK), jnp.int32),
            pltpu.VMEM((CHUNK, 16), jnp.float32),
            pltpu.VMEM_SHARED((N_ACC, 16), jnp.float32),
        ],
        compiler_params=pltpu.CompilerParams(use_tc_tiling_on_sc=False),
    )
    return k(sidx, ones, zeros)


NPIECE = 4
HH = H // NPIECE


NBUF = 4


def _sc_agg_body(g_hbm, gidx_hbm, sidx_hbm, out_hbm, gidx_v, sidx_v,
                 b0, b1, b2, b3, acc, gsrc,
                 gs0, gs1, gs2, gs3, ss0, ss1, ss2, ss3):
    bufs = (b0, b1, b2, b3)
    gsems = (gs0, gs1, gs2, gs3)
    ssems = (ss0, ss1, ss2, ss3)
    c = lax.axis_index("c")
    s = lax.axis_index("s")
    w = c * NUM_TILES + s
    pltpu.sync_copy(gidx_hbm.at[w], gidx_v)
    pltpu.sync_copy(sidx_hbm.at[w], sidx_v)
    base = s * ROWS_PER_TILE

    for half in range(NPIECE):
        pltpu.sync_copy(g_hbm.at[half].at[pl.ds(c * N_PAD + base,
                                                ROWS_PER_TILE)],
                        gsrc.at[pl.ds(base, ROWS_PER_TILE)])
        pltpu.sync_copy(g_hbm.at[half].at[pl.ds(c * N_PAD + base,
                                                ROWS_PER_TILE)],
                        acc.at[pl.ds(base, ROWS_PER_TILE)])
        plsc.subcore_barrier()

        def g_src(j):
            return gsrc.at[gidx_v.at[j]]

        def s_dst(j):
            return acc.at[sidx_v.at[j]]

        for k in range(NBUF):
            pltpu.async_copy(g_src(k), bufs[k], gsems[k])

        @pl.loop(0, CHUNKS // NBUF - 1)
        def _(t):
            j0 = t * NBUF
            for k in range(NBUF):
                pltpu.make_async_copy(g_src(j0 + k), bufs[k], gsems[k]).wait()
                pltpu.async_copy(bufs[k], s_dst(j0 + k), ssems[k], add=True)
            for k in range(NBUF):
                pltpu.make_async_copy(bufs[k], s_dst(j0 + k), ssems[k]).wait()
                pltpu.async_copy(g_src(j0 + NBUF + k), bufs[k], gsems[k])

        j0 = CHUNKS - NBUF
        for k in range(NBUF):
            pltpu.make_async_copy(g_src(j0 + k), bufs[k], gsems[k]).wait()
            pltpu.async_copy(bufs[k], s_dst(j0 + k), ssems[k], add=True)
        for k in range(NBUF):
            pltpu.make_async_copy(bufs[k], s_dst(j0 + k), ssems[k]).wait()

        plsc.subcore_barrier()
        pltpu.sync_copy(acc.at[pl.ds(base, ROWS_PER_TILE)],
                        out_hbm.at[half].at[pl.ds(c * N_PAD + base,
                                                  ROWS_PER_TILE)])


def _sc_agg(g, gidx, sidx):
    k = pl.kernel(
        _sc_agg_body,
        out_type=jax.ShapeDtypeStruct((NPIECE, NUM_SC * N_PAD, HH),
                                      jnp.float32),
        mesh=_MESH,
        scratch_types=[
            pltpu.VMEM((CHUNKS, CHUNK), jnp.int32),
            pltpu.VMEM((CHUNKS, CHUNK), jnp.int32),
        ] + [pltpu.VMEM((CHUNK, HH), jnp.float32)] * NBUF + [
            pltpu.VMEM_SHARED((N_ACC, HH), jnp.float32),
            pltpu.VMEM_SHARED((N_PAD, HH), jnp.float32),
        ] + [pltpu.SemaphoreType.DMA] * (2 * NBUF),
        compiler_params=pltpu.CompilerParams(use_tc_tiling_on_sc=False),
    )
    return k(g, gidx, sidx)


def _to_halves(gp):
    return gp.reshape(NUM_SC, N_PAD, NPIECE, HH).transpose(2, 0, 1, 3).reshape(
        NPIECE, NUM_SC * N_PAD, HH)


def _from_halves(o):
    return o.reshape(NPIECE, NUM_SC, N_PAD, HH).transpose(1, 2, 0, 3).reshape(
        NUM_SC, N_PAD, H)[:, :N]


def _mm_shared_body(a_ref, w_ref, o_ref):
    o_ref[...] = jnp.dot(a_ref[...], w_ref[0],
                         preferred_element_type=jnp.float32)[None]


def _mm_shared(x, wstack):
    return pl.pallas_call(
        _mm_shared_body,
        grid=(NUM_SC, N // MM_BLK),
        in_specs=[
            pl.BlockSpec((MM_BLK, H), lambda d, i: (i, 0)),
            pl.BlockSpec((1, H, H), lambda d, i: (d, 0, 0)),
        ],
        out_specs=pl.BlockSpec((1, MM_BLK, H), lambda d, i: (d, i, 0)),
        out_shape=jax.ShapeDtypeStruct((NUM_SC, N, H), jnp.float32),
    )(x, wstack)


def _mm_stacked_body(a_ref, w_ref, o_ref):
    o_ref[...] = jnp.dot(a_ref[0], w_ref[0],
                         preferred_element_type=jnp.float32)[None]


def _mm_stacked(a, wstack):
    return pl.pallas_call(
        _mm_stacked_body,
        grid=(NUM_SC, N // MM_BLK),
        in_specs=[
            pl.BlockSpec((1, MM_BLK, H), lambda d, i: (d, i, 0)),
            pl.BlockSpec((1, H, H), lambda d, i: (d, 0, 0)),
        ],
        out_specs=pl.BlockSpec((1, MM_BLK, H), lambda d, i: (d, i, 0)),
        out_shape=jax.ShapeDtypeStruct((NUM_SC, N, H), jnp.float32),
    )(a, wstack)


def _fc_logsoftmax_body(xc_ref, wc_ref, bfc_ref, out_ref):
    l = jnp.dot(xc_ref[...], wc_ref[...], preferred_element_type=jnp.float32)
    l = l + bfc_ref[...]
    m = jnp.max(l, axis=1, keepdims=True)
    lse = m + jnp.log(jnp.sum(jnp.exp(l - m), axis=1, keepdims=True))
    out_ref[...] = l - lse


def _fc_logsoftmax(xf, xb, Wfc, bfc):
    xc = jnp.concatenate([xf, xb], axis=1)
    wc = Wfc.T
    b2 = bfc.reshape(1, N)
    return pl.pallas_call(
        _fc_logsoftmax_body,
        grid=(N // ROW_BLK,),
        in_specs=[
            pl.BlockSpec((ROW_BLK, 2 * H), lambda i: (i, 0)),
            pl.BlockSpec((2 * H, N), lambda i: (0, 0)),
            pl.BlockSpec((1, N), lambda i: (0, 0)),
        ],
        out_specs=pl.BlockSpec((ROW_BLK, N), lambda i: (i, 0)),
        out_shape=jax.ShapeDtypeStruct((N, N), jnp.float32),
    )(xc, wc, b2)


def kernel(x, edge_index, W1f, b1f, W2f, b2f, W1b, b1b, W2b, b2b, Wfc, bfc):
    src = edge_index[0]
    dst = edge_index[1]

    pad = E_PAD - E
    padz = jnp.zeros((pad,), jnp.int32)
    padn = jnp.full((pad,), N, jnp.int32)
    gf = jnp.concatenate([src, padz])
    sf = jnp.concatenate([dst, padn])
    gb = jnp.concatenate([dst, padz])
    sb = jnp.concatenate([src, padn])
    gidx = jnp.stack([gf, gb]).reshape(NUM_SC * NUM_TILES, CHUNKS, CHUNK)
    sidx = jnp.stack([sf, sb]).reshape(NUM_SC * NUM_TILES, CHUNKS, CHUNK)

    ones = jnp.ones((CHUNK, 16), jnp.float32)
    zeros = jnp.zeros((N_PAD, 16), jnp.float32)
    degp = _sc_deg(sidx, ones, zeros)
    dinv = lax.rsqrt(degp[:, :N, 0] + 1.0)
    dinv3 = dinv[:, :, None]

    wstack1 = jnp.stack([W1f.T, W1b.T])
    wstack2 = jnp.stack([W2f.T, W2b.T])
    bstack1 = jnp.stack([b1f, b1b])[:, None, :]
    bstack2 = jnp.stack([b2f, b2b])[:, None, :]

    rowpad = ((0, 0), (0, N_PAD - N), (0, 0))
    h1 = _mm_shared(x, wstack1)
    g1 = _to_halves(jnp.pad(h1 * dinv3, rowpad))
    agg1 = _from_halves(_sc_agg(g1, gidx, sidx))
    x1 = jax.nn.relu(agg1 * dinv3 + bstack1)

    h2 = _mm_stacked(x1, wstack2)
    g2 = _to_halves(jnp.pad(h2 * dinv3, rowpad))
    agg2 = _from_halves(_sc_agg(g2, gidx, sidx))
    x2 = jax.nn.relu(agg2 * dinv3 + bstack2)

    return _fc_logsoftmax(x2[0], x2[1], Wfc, bfc)
